# Initial kernel scaffold; baseline (speedup 1.0000x reference)
import jax
import jax.numpy as jnp
from jax.experimental import pallas as pl


def kernel(features, edge_index_r0, edge_index_r1, edge_index_r2, W_gcn, b_gcn, W_nn1, b_nn1, W_out, b_out):
    raise NotImplementedError("write your pallas kernel here")


# speedup vs baseline: 6.2061x; 6.2061x over previous
import functools

import jax
import jax.numpy as jnp
from jax import lax
from jax.experimental import pallas as pl
from jax.experimental.pallas import tpu as pltpu
from jax.experimental.pallas import tpu_sc as plsc

N = 10000
E = 320000
D = 128
H = 128
R = 3

PAD_N = 10240
NB = PAD_N // 128
HB = 1024
NC = 2
NS = 16
CHUNK = 80
BPB = 5
EPT = E // (NC * NS)
NBLK = EPT // (BPB * CHUNK)


def _mm_body(x_ref, w_ref, o0_ref, o1_ref, o2_ref):
    x = x_ref[...]
    o0_ref[...] = jnp.dot(x, w_ref[0], preferred_element_type=jnp.float32)
    o1_ref[...] = jnp.dot(x, w_ref[1], preferred_element_type=jnp.float32)
    o2_ref[...] = jnp.dot(x, w_ref[2], preferred_element_type=jnp.float32)


def _tc_matmul(xpad, W_gcn):
    out = jax.ShapeDtypeStruct((PAD_N, D), jnp.float32)
    return pl.pallas_call(
        _mm_body,
        grid=(NB,),
        in_specs=[
            pl.BlockSpec((128, D), lambda i: (i, 0)),
            pl.BlockSpec((R, D, H), lambda i: (0, 0, 0)),
        ],
        out_specs=[
            pl.BlockSpec((128, H), lambda i: (i, 0)),
            pl.BlockSpec((128, H), lambda i: (i, 0)),
            pl.BlockSpec((128, H), lambda i: (i, 0)),
        ],
        out_shape=[out, out, out],
    )(xpad, W_gcn)


def _sc_body(xm0, xm1, xm2, ei0, ei1, ei2, aggp, degp,
             src_blk, dst_blk, rows, hist, zeros, sem, agg_sh):
    cid = lax.axis_index("c")
    sid = lax.axis_index("s")
    wid = cid * NS + sid
    zv = jnp.zeros((16,), jnp.float32)
    ones = jnp.ones((16,), jnp.float32)

    def _zrow(i, _):
        zeros[i // 8, pl.ds((i % 8) * 16, 16)] = zv
        return 0
    lax.fori_loop(0, (64 * 128) // 16, _zrow, 0)

    def _zstripe(z, _):
        pltpu.sync_copy(zeros, agg_sh.at[pl.ds(sid * 640 + z * 64, 64)])
        return 0
    lax.fori_loop(0, 10, _zstripe, 0)
    plsc.subcore_barrier()

    xms = (xm0, xm1, xm2)
    eis = (ei0, ei1, ei2)
    for r in range(R):
        def _zhist(i, _):
            hist[pl.ds(i * 16, 16)] = zv
            return 0
        lax.fori_loop(0, PAD_N // 16, _zhist, 0)

        def _blk(ib, _):
            pltpu.sync_copy(eis[r].at[0, wid, ib], src_blk)
            pltpu.sync_copy(eis[r].at[1, wid, ib], dst_blk)

            def _chunk(j, _):
                pltpu.async_copy(xms[r].at[src_blk.at[j]], rows, sem).wait()
                def _hist(k, _):
                    dv = dst_blk[j, pl.ds(k * 16, 16)]
                    plsc.addupdate_scatter(hist, [dv], ones)
                    return 0
                lax.fori_loop(0, CHUNK // 16, _hist, 0)
                pltpu.sync_copy(rows, agg_sh.at[dst_blk.at[j]], add=True)
                return 0
            lax.fori_loop(0, BPB, _chunk, 0)
            return 0
        lax.fori_loop(0, NBLK, _blk, 0)

        plsc.subcore_barrier()
        pltpu.sync_copy(agg_sh.at[pl.ds(sid * 640, 640)],
                        aggp.at[cid, r, pl.ds(sid * 640, 640)])
        pltpu.sync_copy(hist, degp.at[cid, sid, pl.ds(r * PAD_N, PAD_N)])
        if r < R - 1:
            lax.fori_loop(0, 10, _zstripe, 0)
            plsc.subcore_barrier()


def _sc_aggregate(xm0, xm1, xm2, ei0, ei1, ei2):
    mesh = plsc.VectorSubcoreMesh(core_axis_name="c", subcore_axis_name="s")
    kern = pl.kernel(
        _sc_body,
        out_type=(
            jax.ShapeDtypeStruct((NC, R, PAD_N, H), jnp.float32),
            jax.ShapeDtypeStruct((NC, NS, R * PAD_N), jnp.float32),
        ),
        mesh=mesh,
        compiler_params=pltpu.CompilerParams(needs_layout_passes=False),
        scratch_types=[
            pltpu.VMEM((BPB, CHUNK), jnp.int32),
            pltpu.VMEM((BPB, CHUNK), jnp.int32),
            pltpu.VMEM((CHUNK, H), jnp.float32),
            pltpu.VMEM((PAD_N,), jnp.float32),
            pltpu.VMEM((64, 128), jnp.float32),
            pltpu.SemaphoreType.DMA,
            pltpu.VMEM_SHARED((PAD_N, H), jnp.float32),
        ],
    )
    return kern(xm0, xm1, xm2, ei0, ei1, ei2)


def _head_body(aggp_ref, degp_ref, bg_ref, w1_ref, b1_ref, wo_ref, bo_ref,
               out_ref):
    deg = jnp.sum(degp_ref[...], axis=(0, 1))
    h1 = jnp.zeros((HB, H), jnp.float32)
    for r in range(R):
        a = aggp_ref[0, r] + aggp_ref[1, r]
        w = 1.0 / jnp.maximum(deg[r].reshape(HB), 1.0)
        h1 = h1 + a * w[:, None] + bg_ref[r][None, :]
    h2 = jnp.maximum(h1, 0.0)
    z = jnp.dot(h2, w1_ref[...], preferred_element_type=jnp.float32)
    z = z + b1_ref[0][None, :]
    h3 = 1.0 / (1.0 + jnp.exp(-z))
    o = jnp.dot(h3, wo_ref[...], preferred_element_type=jnp.float32)
    out_ref[...] = o + bo_ref[0][None, :]


def _tc_head(aggp, degp5, b_gcn, W_nn1, b_nn1, W_out_p, b_out_p):
    return pl.pallas_call(
        _head_body,
        grid=(PAD_N // HB,),
        in_specs=[
            pl.BlockSpec((NC, R, HB, H), lambda i: (0, 0, i, 0)),
            pl.BlockSpec((NC, NS, R, HB // 128, 128), lambda i: (0, 0, 0, i, 0)),
            pl.BlockSpec((R, H), lambda i: (0, 0)),
            pl.BlockSpec((H, H), lambda i: (0, 0)),
            pl.BlockSpec((1, H), lambda i: (0, 0)),
            pl.BlockSpec((H, 128), lambda i: (0, 0)),
            pl.BlockSpec((1, 128), lambda i: (0, 0)),
        ],
        out_specs=pl.BlockSpec((HB, 128), lambda i: (i, 0)),
        out_shape=jax.ShapeDtypeStruct((PAD_N, 128), jnp.float32),
    )(aggp, degp5, b_gcn, W_nn1, b_nn1, W_out_p, b_out_p)


def kernel(features, edge_index_r0, edge_index_r1, edge_index_r2,
           W_gcn, b_gcn, W_nn1, b_nn1, W_out, b_out):
    xpad = jnp.pad(features, ((0, PAD_N - N), (0, 0)))
    xm0, xm1, xm2 = _tc_matmul(xpad, W_gcn)

    ei0 = edge_index_r0.reshape(2, NC * NS, NBLK, BPB, CHUNK)
    ei1 = edge_index_r1.reshape(2, NC * NS, NBLK, BPB, CHUNK)
    ei2 = edge_index_r2.reshape(2, NC * NS, NBLK, BPB, CHUNK)
    aggp, degp = _sc_aggregate(xm0, xm1, xm2, ei0, ei1, ei2)

    degp5 = degp.reshape(NC, NS, R, NB, 128)
    W_out_p = jnp.zeros((H, 128), jnp.float32).at[:, :2].set(W_out)
    b_out_p = jnp.zeros((1, 128), jnp.float32).at[0, :2].set(b_out)
    out = _tc_head(aggp, degp5, b_gcn, W_nn1, b_nn1.reshape(1, H),
                   W_out_p, b_out_p)
    return out[:N, :2]


# speedup vs baseline: 8.0611x; 1.2989x over previous
import functools

import jax
import jax.numpy as jnp
from jax import lax
from jax.experimental import pallas as pl
from jax.experimental.pallas import tpu as pltpu
from jax.experimental.pallas import tpu_sc as plsc

N = 10000
E = 320000
D = 128
H = 128
R = 3

PAD_N = 10240
NB = PAD_N // 128
HB = 1024
NC = 2
NS = 16
CHUNK = 80
BPB = 5
EPT = E // (NC * NS)
NBLK = EPT // (BPB * CHUNK)


def _mm_body(x_ref, w_ref, o0_ref, o1_ref, o2_ref):
    x = x_ref[...]
    o0_ref[...] = jnp.dot(x, w_ref[0], preferred_element_type=jnp.float32)
    o1_ref[...] = jnp.dot(x, w_ref[1], preferred_element_type=jnp.float32)
    o2_ref[...] = jnp.dot(x, w_ref[2], preferred_element_type=jnp.float32)


def _tc_matmul(xpad, W_gcn):
    out = jax.ShapeDtypeStruct((PAD_N, D), jnp.float32)
    return pl.pallas_call(
        _mm_body,
        grid=(NB,),
        in_specs=[
            pl.BlockSpec((128, D), lambda i: (i, 0)),
            pl.BlockSpec((R, D, H), lambda i: (0, 0, 0)),
        ],
        out_specs=[
            pl.BlockSpec((128, H), lambda i: (i, 0)),
            pl.BlockSpec((128, H), lambda i: (i, 0)),
            pl.BlockSpec((128, H), lambda i: (i, 0)),
        ],
        out_shape=[out, out, out],
    )(xpad, W_gcn)


def _sc_body(xm0, xm1, xm2, ei0, ei1, ei2, aggp, degp,
             src_blk, dst_blk, rows_a, rows_b, hist, zeros,
             gsem_a, gsem_b, ssem_a, ssem_b, agg_sh):
    cid = lax.axis_index("c")
    sid = lax.axis_index("s")
    wid = cid * NS + sid
    zv = jnp.zeros((16,), jnp.float32)
    ones = jnp.ones((16,), jnp.float32)

    def _zrow(i, _):
        zeros[i // 8, pl.ds((i % 8) * 16, 16)] = zv
        return 0
    lax.fori_loop(0, (64 * 128) // 16, _zrow, 0)

    def _zstripe(z, _):
        pltpu.sync_copy(zeros, agg_sh.at[pl.ds(sid * 640 + z * 64, 64)])
        return 0
    lax.fori_loop(0, 10, _zstripe, 0)
    plsc.subcore_barrier()

    xms = (xm0, xm1, xm2)
    eis = (ei0, ei1, ei2)
    for r in range(R):
        def _zhist(i, _):
            hist[pl.ds(i * 16, 16)] = zv
            return 0
        lax.fori_loop(0, PAD_N // 16, _zhist, 0)

        def _blk(ib, _):
            pltpu.sync_copy(eis[r].at[0, wid, ib], src_blk)
            pltpu.sync_copy(eis[r].at[1, wid, ib], dst_blk)

            bufs = (rows_a, rows_b)
            gsems = (gsem_a, gsem_b)
            ssems = (ssem_a, ssem_b)
            sg = [None] * BPB
            ss = [None] * BPB
            sg[0] = pltpu.async_copy(xms[r].at[src_blk.at[0]], bufs[0],
                                     gsems[0])
            sg[1] = pltpu.async_copy(xms[r].at[src_blk.at[1]], bufs[1],
                                     gsems[1])
            for j in range(BPB):
                p = j % 2
                sg[j].wait()
                def _hist(k, _, _j=j):
                    dv = dst_blk[_j, pl.ds(k * 16, 16)]
                    plsc.addupdate_scatter(hist, [dv], ones)
                    return 0
                lax.fori_loop(0, CHUNK // 16, _hist, 0)
                ss[j] = pltpu.async_copy(bufs[p], agg_sh.at[dst_blk.at[j]],
                                         ssems[p], add=True)
                if j + 2 < BPB:
                    ss[j].wait()
                    sg[j + 2] = pltpu.async_copy(
                        xms[r].at[src_blk.at[j + 2]], bufs[p], gsems[p])
            ss[BPB - 2].wait()
            ss[BPB - 1].wait()
            return 0
        lax.fori_loop(0, NBLK, _blk, 0)

        plsc.subcore_barrier()
        pltpu.sync_copy(agg_sh.at[pl.ds(sid * 640, 640)],
                        aggp.at[cid, r, pl.ds(sid * 640, 640)])
        pltpu.sync_copy(hist, degp.at[cid, sid, pl.ds(r * PAD_N, PAD_N)])
        if r < R - 1:
            lax.fori_loop(0, 10, _zstripe, 0)
            plsc.subcore_barrier()


def _sc_aggregate(xm0, xm1, xm2, ei0, ei1, ei2):
    mesh = plsc.VectorSubcoreMesh(core_axis_name="c", subcore_axis_name="s")
    kern = pl.kernel(
        _sc_body,
        out_type=(
            jax.ShapeDtypeStruct((NC, R, PAD_N, H), jnp.float32),
            jax.ShapeDtypeStruct((NC, NS, R * PAD_N), jnp.float32),
        ),
        mesh=mesh,
        compiler_params=pltpu.CompilerParams(needs_layout_passes=False),
        scratch_types=[
            pltpu.VMEM((BPB, CHUNK), jnp.int32),
            pltpu.VMEM((BPB, CHUNK), jnp.int32),
            pltpu.VMEM((CHUNK, H), jnp.float32),
            pltpu.VMEM((CHUNK, H), jnp.float32),
            pltpu.VMEM((PAD_N,), jnp.float32),
            pltpu.VMEM((64, 128), jnp.float32),
            pltpu.SemaphoreType.DMA,
            pltpu.SemaphoreType.DMA,
            pltpu.SemaphoreType.DMA,
            pltpu.SemaphoreType.DMA,
            pltpu.VMEM_SHARED((PAD_N, H), jnp.float32),
        ],
    )
    return kern(xm0, xm1, xm2, ei0, ei1, ei2)


def _head_body(aggp_ref, degp_ref, bg_ref, w1_ref, b1_ref, wo_ref, bo_ref,
               out_ref):
    deg = jnp.sum(degp_ref[...], axis=(0, 1))
    h1 = jnp.zeros((HB, H), jnp.float32)
    for r in range(R):
        a = aggp_ref[0, r] + aggp_ref[1, r]
        w = 1.0 / jnp.maximum(deg[r].reshape(HB), 1.0)
        h1 = h1 + a * w[:, None] + bg_ref[r][None, :]
    h2 = jnp.maximum(h1, 0.0)
    z = jnp.dot(h2, w1_ref[...], preferred_element_type=jnp.float32)
    z = z + b1_ref[0][None, :]
    h3 = 1.0 / (1.0 + jnp.exp(-z))
    o = jnp.dot(h3, wo_ref[...], preferred_element_type=jnp.float32)
    out_ref[...] = o + bo_ref[0][None, :]


def _tc_head(aggp, degp5, b_gcn, W_nn1, b_nn1, W_out_p, b_out_p):
    return pl.pallas_call(
        _head_body,
        grid=(PAD_N // HB,),
        in_specs=[
            pl.BlockSpec((NC, R, HB, H), lambda i: (0, 0, i, 0)),
            pl.BlockSpec((NC, NS, R, HB // 128, 128), lambda i: (0, 0, 0, i, 0)),
            pl.BlockSpec((R, H), lambda i: (0, 0)),
            pl.BlockSpec((H, H), lambda i: (0, 0)),
            pl.BlockSpec((1, H), lambda i: (0, 0)),
            pl.BlockSpec((H, 128), lambda i: (0, 0)),
            pl.BlockSpec((1, 128), lambda i: (0, 0)),
        ],
        out_specs=pl.BlockSpec((HB, 128), lambda i: (i, 0)),
        out_shape=jax.ShapeDtypeStruct((PAD_N, 128), jnp.float32),
    )(aggp, degp5, b_gcn, W_nn1, b_nn1, W_out_p, b_out_p)


def kernel(features, edge_index_r0, edge_index_r1, edge_index_r2,
           W_gcn, b_gcn, W_nn1, b_nn1, W_out, b_out):
    xpad = jnp.pad(features, ((0, PAD_N - N), (0, 0)))
    xm0, xm1, xm2 = _tc_matmul(xpad, W_gcn)

    ei0 = edge_index_r0.reshape(2, NC * NS, NBLK, BPB, CHUNK)
    ei1 = edge_index_r1.reshape(2, NC * NS, NBLK, BPB, CHUNK)
    ei2 = edge_index_r2.reshape(2, NC * NS, NBLK, BPB, CHUNK)
    aggp, degp = _sc_aggregate(xm0, xm1, xm2, ei0, ei1, ei2)

    degp5 = degp.reshape(NC, NS, R, NB, 128)
    W_out_p = jnp.zeros((H, 128), jnp.float32).at[:, :2].set(W_out)
    b_out_p = jnp.zeros((1, 128), jnp.float32).at[0, :2].set(b_out)
    out = _tc_head(aggp, degp5, b_gcn, W_nn1, b_nn1.reshape(1, H),
                   W_out_p, b_out_p)
    return out[:N, :2]


# speedup vs baseline: 8.6740x; 1.0760x over previous
import jax
import jax.numpy as jnp
from jax import lax
from jax.experimental import pallas as pl
from jax.experimental.pallas import tpu as pltpu
from jax.experimental.pallas import tpu_sc as plsc

N = 10000
E = 320000
D = 128
H = 128
R = 3

PAD_N = 10240
NB = PAD_N // 128
HB = 1024
NC = 2
NS = 16
CHUNK = 80
HALF = CHUNK // 2
BPB = 5
EPT = E // (NC * NS)
NBLK = EPT // (BPB * CHUNK)
NBUF = 3
STRIPE = PAD_N // NS


def _mm_body(x_ref, w_ref, o0_ref, o1_ref, o2_ref):
    x = x_ref[...]
    o0_ref[...] = jnp.dot(x, w_ref[0], preferred_element_type=jnp.float32)
    o1_ref[...] = jnp.dot(x, w_ref[1], preferred_element_type=jnp.float32)
    o2_ref[...] = jnp.dot(x, w_ref[2], preferred_element_type=jnp.float32)


def _tc_matmul(xpad, W_gcn):
    out = jax.ShapeDtypeStruct((PAD_N, D), jnp.float32)
    return pl.pallas_call(
        _mm_body,
        grid=(NB,),
        in_specs=[
            pl.BlockSpec((128, D), lambda i: (i, 0)),
            pl.BlockSpec((R, D, H), lambda i: (0, 0, 0)),
        ],
        out_specs=[
            pl.BlockSpec((128, H), lambda i: (i, 0)),
            pl.BlockSpec((128, H), lambda i: (i, 0)),
            pl.BlockSpec((128, H), lambda i: (i, 0)),
        ],
        out_shape=[out, out, out],
    )(xpad, W_gcn)


def _sc_body(xm0, xm1, xm2, ei0, ei1, ei2, aggp, degp,
             src_blk, dst_blk, rows0, rows1, rows2, hist,
             ga0, gb0, ga1, gb1, ga2, gb2, ssem0, ssem1, ssem2,
             agg_sh):
    cid = lax.axis_index("c")
    sid = lax.axis_index("s")
    wid = cid * NS + sid
    zv = jnp.zeros((16,), jnp.float32)
    ones = jnp.ones((16,), jnp.float32)
    bufs = (rows0, rows1, rows2)
    gsems = ((ga0, gb0), (ga1, gb1), (ga2, gb2))
    ssems = (ssem0, ssem1, ssem2)

    def _zbuf(i, _):
        rows0[i // 8, pl.ds((i % 8) * 16, 16)] = zv
        return 0

    def _zhist(i, _):
        hist[pl.ds(i * 16, 16)] = zv
        return 0

    def _zstripe(z, _):
        pltpu.sync_copy(rows0, agg_sh.at[pl.ds(sid * STRIPE + z * CHUNK,
                                               CHUNK)])
        return 0

    xms = (xm0, xm1, xm2)
    eis = (ei0, ei1, ei2)

    def _gather(r, j, buf):
        a = pltpu.async_copy(xms[r].at[src_blk.at[j, pl.ds(0, HALF)]],
                             buf.at[pl.ds(0, HALF)], gsems[j % NBUF][0])
        b = pltpu.async_copy(xms[r].at[src_blk.at[j, pl.ds(HALF, HALF)]],
                             buf.at[pl.ds(HALF, HALF)], gsems[j % NBUF][1])
        return (a, b)

    for r in range(R):
        lax.fori_loop(0, (CHUNK * H) // 16, _zbuf, 0)
        lax.fori_loop(0, STRIPE // CHUNK, _zstripe, 0)
        lax.fori_loop(0, PAD_N // 16, _zhist, 0)
        plsc.subcore_barrier()

        def _blk(ib, _):
            pltpu.sync_copy(eis[r].at[0, wid, ib], src_blk)
            pltpu.sync_copy(eis[r].at[1, wid, ib], dst_blk)
            sg = [None] * BPB
            ss = [None] * BPB
            sg[0] = _gather(r, 0, bufs[0])
            sg[1] = _gather(r, 1, bufs[1])
            for j in range(BPB):
                p = j % NBUF
                sg[j][0].wait()
                sg[j][1].wait()
                for k in range(CHUNK // 16):
                    dv = dst_blk[j, pl.ds(k * 16, 16)]
                    plsc.addupdate_scatter(hist, [dv], ones)
                ss[j] = pltpu.async_copy(bufs[p], agg_sh.at[dst_blk.at[j]],
                                         ssems[p], add=True)
                if j >= 1:
                    ss[j - 1].wait()
                if j + 2 < BPB:
                    sg[j + 2] = _gather(r, j + 2, bufs[(j + 2) % NBUF])
            ss[BPB - 1].wait()
            return 0
        lax.fori_loop(0, NBLK, _blk, 0)

        plsc.subcore_barrier()
        pltpu.sync_copy(agg_sh.at[pl.ds(sid * STRIPE, STRIPE)],
                        aggp.at[cid, r, pl.ds(sid * STRIPE, STRIPE)])
        pltpu.sync_copy(hist, degp.at[cid, sid, pl.ds(r * PAD_N, PAD_N)])


def _sc_aggregate(xm0, xm1, xm2, ei0, ei1, ei2):
    mesh = plsc.VectorSubcoreMesh(core_axis_name="c", subcore_axis_name="s")
    kern = pl.kernel(
        _sc_body,
        out_type=(
            jax.ShapeDtypeStruct((NC, R, PAD_N, H), jnp.float32),
            jax.ShapeDtypeStruct((NC, NS, R * PAD_N), jnp.float32),
        ),
        mesh=mesh,
        compiler_params=pltpu.CompilerParams(needs_layout_passes=False),
        scratch_types=[
            pltpu.VMEM((BPB, CHUNK), jnp.int32),
            pltpu.VMEM((BPB, CHUNK), jnp.int32),
            pltpu.VMEM((CHUNK, H), jnp.float32),
            pltpu.VMEM((CHUNK, H), jnp.float32),
            pltpu.VMEM((CHUNK, H), jnp.float32),
            pltpu.VMEM((PAD_N,), jnp.float32),
            pltpu.SemaphoreType.DMA,
            pltpu.SemaphoreType.DMA,
            pltpu.SemaphoreType.DMA,
            pltpu.SemaphoreType.DMA,
            pltpu.SemaphoreType.DMA,
            pltpu.SemaphoreType.DMA,
            pltpu.SemaphoreType.DMA,
            pltpu.SemaphoreType.DMA,
            pltpu.SemaphoreType.DMA,
            pltpu.VMEM_SHARED((PAD_N, H), jnp.float32),
        ],
    )
    return kern(xm0, xm1, xm2, ei0, ei1, ei2)


def _head_body(aggp_ref, degp_ref, bg_ref, w1_ref, b1_ref, wo_ref, bo_ref,
               out_ref):
    deg = jnp.sum(degp_ref[...], axis=(0, 1))
    h1 = jnp.zeros((HB, H), jnp.float32)
    for r in range(R):
        a = aggp_ref[0, r] + aggp_ref[1, r]
        w = 1.0 / jnp.maximum(deg[r].reshape(HB), 1.0)
        h1 = h1 + a * w[:, None] + bg_ref[r][None, :]
    h2 = jnp.maximum(h1, 0.0)
    z = jnp.dot(h2, w1_ref[...], preferred_element_type=jnp.float32)
    z = z + b1_ref[0][None, :]
    h3 = 1.0 / (1.0 + jnp.exp(-z))
    o = jnp.dot(h3, wo_ref[...], preferred_element_type=jnp.float32)
    out_ref[...] = o + bo_ref[0][None, :]


def _tc_head(aggp, degp5, b_gcn, W_nn1, b_nn1, W_out_p, b_out_p):
    return pl.pallas_call(
        _head_body,
        grid=(PAD_N // HB,),
        in_specs=[
            pl.BlockSpec((NC, R, HB, H), lambda i: (0, 0, i, 0)),
            pl.BlockSpec((NC, NS, R, HB // 128, 128), lambda i: (0, 0, 0, i, 0)),
            pl.BlockSpec((R, H), lambda i: (0, 0)),
            pl.BlockSpec((H, H), lambda i: (0, 0)),
            pl.BlockSpec((1, H), lambda i: (0, 0)),
            pl.BlockSpec((H, 128), lambda i: (0, 0)),
            pl.BlockSpec((1, 128), lambda i: (0, 0)),
        ],
        out_specs=pl.BlockSpec((HB, 128), lambda i: (i, 0)),
        out_shape=jax.ShapeDtypeStruct((PAD_N, 128), jnp.float32),
    )(aggp, degp5, b_gcn, W_nn1, b_nn1, W_out_p, b_out_p)


def kernel(features, edge_index_r0, edge_index_r1, edge_index_r2,
           W_gcn, b_gcn, W_nn1, b_nn1, W_out, b_out):
    xpad = jnp.pad(features, ((0, PAD_N - N), (0, 0)))
    xm0, xm1, xm2 = _tc_matmul(xpad, W_gcn)

    ei0 = edge_index_r0.reshape(2, NC * NS, NBLK, BPB, CHUNK)
    ei1 = edge_index_r1.reshape(2, NC * NS, NBLK, BPB, CHUNK)
    ei2 = edge_index_r2.reshape(2, NC * NS, NBLK, BPB, CHUNK)
    aggp, degp = _sc_aggregate(xm0, xm1, xm2, ei0, ei1, ei2)

    degp5 = degp.reshape(NC, NS, R, NB, 128)
    W_out_p = jnp.zeros((H, 128), jnp.float32).at[:, :2].set(W_out)
    b_out_p = jnp.zeros((1, 128), jnp.float32).at[0, :2].set(b_out)
    out = _tc_head(aggp, degp5, b_gcn, W_nn1, b_nn1.reshape(1, H),
                   W_out_p, b_out_p)
    return out[:N, :2]


# speedup vs baseline: 10.0842x; 1.1626x over previous
import jax
import jax.numpy as jnp
from jax import lax
from jax.experimental import pallas as pl
from jax.experimental.pallas import tpu as pltpu
from jax.experimental.pallas import tpu_sc as plsc

N = 10000
E = 320000
D = 128
H = 128
R = 3

PAD_N = 10240
NB = PAD_N // 128
HB = 1024
NC = 2
NS = 16
CHUNK = 80
HALF = CHUNK // 2
BPB = 5
EPT = E // (NC * NS)
NBLK = EPT // (BPB * CHUNK)
NBUF = 3
STRIPE = PAD_N // NS


def _mm_body(x_ref, w_ref, o0_ref, o1_ref, o2_ref):
    x = x_ref[...]
    o0_ref[...] = jnp.dot(x, w_ref[0], preferred_element_type=jnp.float32)
    o1_ref[...] = jnp.dot(x, w_ref[1], preferred_element_type=jnp.float32)
    o2_ref[...] = jnp.dot(x, w_ref[2], preferred_element_type=jnp.float32)


def _tc_matmul(xpad, W_gcn):
    out = jax.ShapeDtypeStruct((PAD_N, D), jnp.float32)
    return pl.pallas_call(
        _mm_body,
        grid=(NB,),
        in_specs=[
            pl.BlockSpec((128, D), lambda i: (i, 0)),
            pl.BlockSpec((R, D, H), lambda i: (0, 0, 0)),
        ],
        out_specs=[
            pl.BlockSpec((128, H), lambda i: (i, 0)),
            pl.BlockSpec((128, H), lambda i: (i, 0)),
            pl.BlockSpec((128, H), lambda i: (i, 0)),
        ],
        out_shape=[out, out, out],
    )(xpad, W_gcn)


def _sc_body(xm0, xm1, xm2, ei0, ei1, ei2, aggp, degp,
             src_a, dst_a, src_b, dst_b, rows0, rows1, rows2, hist,
             ga0, gb0, ga1, gb1, ga2, gb2, ssem0, ssem1, ssem2,
             isem_a, isem_b, agg_sh):
    cid = lax.axis_index("c")
    sid = lax.axis_index("s")
    wid = cid * NS + sid
    zv = jnp.zeros((16,), jnp.float32)
    ones = jnp.ones((16,), jnp.float32)
    bufs = (rows0, rows1, rows2)
    gsems = ((ga0, gb0), (ga1, gb1), (ga2, gb2))
    ssems = (ssem0, ssem1, ssem2)

    def _zbuf(i, _):
        rows0[i // 8, pl.ds((i % 8) * 16, 16)] = zv
        return 0

    def _zhist(i, _):
        hist[pl.ds(i * 16, 16)] = zv
        return 0

    def _zstripe(z, _):
        pltpu.sync_copy(rows0, agg_sh.at[pl.ds(sid * STRIPE + z * CHUNK,
                                               CHUNK)])
        return 0

    xms = (xm0, xm1, xm2)
    eis = (ei0, ei1, ei2)

    def _stage(r, ib, src_blk, dst_blk, isem):
        a = pltpu.async_copy(eis[r].at[0, wid, ib], src_blk, isem)
        b = pltpu.async_copy(eis[r].at[1, wid, ib], dst_blk, isem)
        return (a, b)

    def _gather(r, j, buf, src_blk):
        a = pltpu.async_copy(xms[r].at[src_blk.at[j, pl.ds(0, HALF)]],
                             buf.at[pl.ds(0, HALF)], gsems[j % NBUF][0])
        b = pltpu.async_copy(xms[r].at[src_blk.at[j, pl.ds(HALF, HALF)]],
                             buf.at[pl.ds(HALF, HALF)], gsems[j % NBUF][1])
        return (a, b)

    def _proc(r, src_blk, dst_blk):
        sg = [None] * BPB
        ss = [None] * BPB
        sg[0] = _gather(r, 0, bufs[0], src_blk)
        sg[1] = _gather(r, 1, bufs[1], src_blk)
        for j in range(BPB):
            p = j % NBUF
            sg[j][0].wait()
            sg[j][1].wait()
            for k in range(CHUNK // 16):
                dv = dst_blk[j, pl.ds(k * 16, 16)]
                plsc.addupdate_scatter(hist, [dv], ones)
            ss[j] = pltpu.async_copy(bufs[p], agg_sh.at[dst_blk.at[j]],
                                     ssems[p], add=True)
            if j >= 1:
                ss[j - 1].wait()
            if j + 2 < BPB:
                sg[j + 2] = _gather(r, j + 2, bufs[(j + 2) % NBUF], src_blk)
        ss[BPB - 1].wait()

    def _drain(blk, isem):
        pltpu.make_async_copy(eis[0].at[0, 0, 0], blk, isem).wait()

    for r in range(R):
        lax.fori_loop(0, (CHUNK * H) // 16, _zbuf, 0)
        lax.fori_loop(0, STRIPE // CHUNK, _zstripe, 0)
        lax.fori_loop(0, PAD_N // 16, _zhist, 0)
        plsc.subcore_barrier()

        _stage(r, 0, src_a, dst_a, isem_a)

        def _pair(q, _):
            _drain(src_a, isem_a)
            _drain(dst_a, isem_a)
            _stage(r, 2 * q + 1, src_b, dst_b, isem_b)
            _proc(r, src_a, dst_a)
            _drain(src_b, isem_b)
            _drain(dst_b, isem_b)
            _stage(r, 2 * q + 2, src_a, dst_a, isem_a)
            _proc(r, src_b, dst_b)
            return 0
        lax.fori_loop(0, NBLK // 2, _pair, 0)
        _drain(src_a, isem_a)
        _drain(dst_a, isem_a)
        _proc(r, src_a, dst_a)

        plsc.subcore_barrier()
        pltpu.sync_copy(agg_sh.at[pl.ds(sid * STRIPE, STRIPE)],
                        aggp.at[cid, r, pl.ds(sid * STRIPE, STRIPE)])
        pltpu.sync_copy(hist, degp.at[cid, sid, pl.ds(r * PAD_N, PAD_N)])


def _sc_aggregate(xm0, xm1, xm2, ei0, ei1, ei2):
    mesh = plsc.VectorSubcoreMesh(core_axis_name="c", subcore_axis_name="s")
    kern = pl.kernel(
        _sc_body,
        out_type=(
            jax.ShapeDtypeStruct((NC, R, PAD_N, H), jnp.float32),
            jax.ShapeDtypeStruct((NC, NS, R * PAD_N), jnp.float32),
        ),
        mesh=mesh,
        compiler_params=pltpu.CompilerParams(needs_layout_passes=False),
        scratch_types=[
            pltpu.VMEM((BPB, CHUNK), jnp.int32),
            pltpu.VMEM((BPB, CHUNK), jnp.int32),
            pltpu.VMEM((BPB, CHUNK), jnp.int32),
            pltpu.VMEM((BPB, CHUNK), jnp.int32),
            pltpu.VMEM((CHUNK, H), jnp.float32),
            pltpu.VMEM((CHUNK, H), jnp.float32),
            pltpu.VMEM((CHUNK, H), jnp.float32),
            pltpu.VMEM((PAD_N,), jnp.float32),
            pltpu.SemaphoreType.DMA,
            pltpu.SemaphoreType.DMA,
            pltpu.SemaphoreType.DMA,
            pltpu.SemaphoreType.DMA,
            pltpu.SemaphoreType.DMA,
            pltpu.SemaphoreType.DMA,
            pltpu.SemaphoreType.DMA,
            pltpu.SemaphoreType.DMA,
            pltpu.SemaphoreType.DMA,
            pltpu.SemaphoreType.DMA,
            pltpu.SemaphoreType.DMA,
            pltpu.VMEM_SHARED((PAD_N, H), jnp.float32),
        ],
    )
    return kern(xm0, xm1, xm2, ei0, ei1, ei2)


def _head_body(aggp_ref, degp_ref, bg_ref, w1_ref, b1_ref, wo_ref, bo_ref,
               out_ref):
    deg = jnp.sum(degp_ref[...], axis=(0, 1))
    h1 = jnp.zeros((HB, H), jnp.float32)
    for r in range(R):
        a = aggp_ref[0, r] + aggp_ref[1, r]
        w = 1.0 / jnp.maximum(deg[r].reshape(HB), 1.0)
        h1 = h1 + a * w[:, None] + bg_ref[r][None, :]
    h2 = jnp.maximum(h1, 0.0)
    z = jnp.dot(h2, w1_ref[...], preferred_element_type=jnp.float32)
    z = z + b1_ref[0][None, :]
    h3 = 1.0 / (1.0 + jnp.exp(-z))
    o = jnp.dot(h3, wo_ref[...], preferred_element_type=jnp.float32)
    out_ref[...] = o + bo_ref[0][None, :]


def _tc_head(aggp, degp5, b_gcn, W_nn1, b_nn1, W_out_p, b_out_p):
    return pl.pallas_call(
        _head_body,
        grid=(PAD_N // HB,),
        in_specs=[
            pl.BlockSpec((NC, R, HB, H), lambda i: (0, 0, i, 0)),
            pl.BlockSpec((NC, NS, R, HB // 128, 128), lambda i: (0, 0, 0, i, 0)),
            pl.BlockSpec((R, H), lambda i: (0, 0)),
            pl.BlockSpec((H, H), lambda i: (0, 0)),
            pl.BlockSpec((1, H), lambda i: (0, 0)),
            pl.BlockSpec((H, 128), lambda i: (0, 0)),
            pl.BlockSpec((1, 128), lambda i: (0, 0)),
        ],
        out_specs=pl.BlockSpec((HB, 128), lambda i: (i, 0)),
        out_shape=jax.ShapeDtypeStruct((PAD_N, 128), jnp.float32),
    )(aggp, degp5, b_gcn, W_nn1, b_nn1, W_out_p, b_out_p)


def kernel(features, edge_index_r0, edge_index_r1, edge_index_r2,
           W_gcn, b_gcn, W_nn1, b_nn1, W_out, b_out):
    xpad = jnp.pad(features, ((0, PAD_N - N), (0, 0)))
    xm0, xm1, xm2 = _tc_matmul(xpad, W_gcn)

    ei0 = edge_index_r0.reshape(2, NC * NS, NBLK, BPB, CHUNK)
    ei1 = edge_index_r1.reshape(2, NC * NS, NBLK, BPB, CHUNK)
    ei2 = edge_index_r2.reshape(2, NC * NS, NBLK, BPB, CHUNK)
    aggp, degp = _sc_aggregate(xm0, xm1, xm2, ei0, ei1, ei2)

    degp5 = degp.reshape(NC, NS, R, NB, 128)
    W_out_p = jnp.zeros((H, 128), jnp.float32).at[:, :2].set(W_out)
    b_out_p = jnp.zeros((1, 128), jnp.float32).at[0, :2].set(b_out)
    out = _tc_head(aggp, degp5, b_gcn, W_nn1, b_nn1.reshape(1, H),
                   W_out_p, b_out_p)
    return out[:N, :2]


# speedup vs baseline: 10.1838x; 1.0099x over previous
import jax
import jax.numpy as jnp
from jax import lax
from jax.experimental import pallas as pl
from jax.experimental.pallas import tpu as pltpu
from jax.experimental.pallas import tpu_sc as plsc

N = 10000
E = 320000
D = 128
H = 128
R = 3

PAD_N = 10240
NB = PAD_N // 128
HB = 1024
NC = 2
NS = 16
CHUNK = 80
HALF = CHUNK // 2
BPB = 5
EPT = E // (NC * NS)
NBLK = EPT // (BPB * CHUNK)
NBUF = 3
STRIPE = PAD_N // NS


def _mm_body(x_ref, w_ref, o0_ref, o1_ref, o2_ref):
    x = x_ref[...]
    o0_ref[...] = jnp.dot(x, w_ref[0], preferred_element_type=jnp.float32)
    o1_ref[...] = jnp.dot(x, w_ref[1], preferred_element_type=jnp.float32)
    o2_ref[...] = jnp.dot(x, w_ref[2], preferred_element_type=jnp.float32)


def _tc_matmul(xpad, W_gcn):
    out = jax.ShapeDtypeStruct((PAD_N, D), jnp.float32)
    return pl.pallas_call(
        _mm_body,
        grid=(NB,),
        in_specs=[
            pl.BlockSpec((128, D), lambda i: (i, 0)),
            pl.BlockSpec((R, D, H), lambda i: (0, 0, 0)),
        ],
        out_specs=[
            pl.BlockSpec((128, H), lambda i: (i, 0)),
            pl.BlockSpec((128, H), lambda i: (i, 0)),
            pl.BlockSpec((128, H), lambda i: (i, 0)),
        ],
        out_shape=[out, out, out],
    )(xpad, W_gcn)


def _sc_body(xm0, xm1, xm2, ei0, ei1, ei2, aggp, degp,
             src_a, dst_a, src_b, dst_b, rows0, rows1, rows2, hist,
             ga0, gb0, ga1, gb1, ga2, gb2, ssem0, ssem1, ssem2,
             isem_a, isem_b, agg_sh):
    cid = lax.axis_index("c")
    sid = lax.axis_index("s")
    wid = cid * NS + sid
    zv = jnp.zeros((16,), jnp.float32)
    ones = jnp.ones((16,), jnp.float32)
    bufs = (rows0, rows1, rows2)
    gsems = ((ga0, gb0), (ga1, gb1), (ga2, gb2))
    ssems = (ssem0, ssem1, ssem2)

    def _zbuf(i, _):
        rows0[i // 8, pl.ds((i % 8) * 16, 16)] = zv
        return 0

    def _zhist(i, _):
        hist[pl.ds(i * 16, 16)] = zv
        return 0

    def _zstripe(z, _):
        pltpu.sync_copy(rows0, agg_sh.at[pl.ds(sid * STRIPE + z * CHUNK,
                                               CHUNK)])
        return 0

    xms = (xm0, xm1, xm2)
    eis = (ei0, ei1, ei2)

    def _stage(r, ib, src_blk, dst_blk, isem):
        a = pltpu.async_copy(eis[r].at[0, wid, ib], src_blk, isem)
        b = pltpu.async_copy(eis[r].at[1, wid, ib], dst_blk, isem)
        return (a, b)

    def _gather(r, j, buf, src_blk):
        a = pltpu.async_copy(xms[r].at[src_blk.at[j, pl.ds(0, HALF)]],
                             buf.at[pl.ds(0, HALF)], gsems[j % NBUF][0])
        b = pltpu.async_copy(xms[r].at[src_blk.at[j, pl.ds(HALF, HALF)]],
                             buf.at[pl.ds(HALF, HALF)], gsems[j % NBUF][1])
        return (a, b)

    def _proc(r, src_blk, dst_blk):
        sg = [None] * BPB
        ss = [None] * BPB
        sg[0] = _gather(r, 0, bufs[0], src_blk)
        sg[1] = _gather(r, 1, bufs[1], src_blk)
        for j in range(BPB):
            p = j % NBUF
            sg[j][0].wait()
            sg[j][1].wait()
            ss[j] = pltpu.async_copy(bufs[p], agg_sh.at[dst_blk.at[j]],
                                     ssems[p], add=True)
            for k in range(CHUNK // 16):
                dv = dst_blk[j, pl.ds(k * 16, 16)]
                plsc.addupdate_scatter(hist, [dv], ones)
            if j >= 1:
                ss[j - 1].wait()
            if j + 2 < BPB:
                sg[j + 2] = _gather(r, j + 2, bufs[(j + 2) % NBUF], src_blk)
        ss[BPB - 1].wait()

    def _drain(blk, isem):
        pltpu.make_async_copy(eis[0].at[0, 0, 0], blk, isem).wait()

    for r in range(R):
        lax.fori_loop(0, (CHUNK * H) // 16, _zbuf, 0)
        lax.fori_loop(0, STRIPE // CHUNK, _zstripe, 0)
        lax.fori_loop(0, PAD_N // 16, _zhist, 0)
        plsc.subcore_barrier()

        _stage(r, 0, src_a, dst_a, isem_a)

        def _pair(q, _):
            _drain(src_a, isem_a)
            _drain(dst_a, isem_a)
            _stage(r, 2 * q + 1, src_b, dst_b, isem_b)
            _proc(r, src_a, dst_a)
            _drain(src_b, isem_b)
            _drain(dst_b, isem_b)
            _stage(r, 2 * q + 2, src_a, dst_a, isem_a)
            _proc(r, src_b, dst_b)
            return 0
        lax.fori_loop(0, NBLK // 2, _pair, 0)
        _drain(src_a, isem_a)
        _drain(dst_a, isem_a)
        _proc(r, src_a, dst_a)

        plsc.subcore_barrier()
        pltpu.sync_copy(agg_sh.at[pl.ds(sid * STRIPE, STRIPE)],
                        aggp.at[cid, r, pl.ds(sid * STRIPE, STRIPE)])
        pltpu.sync_copy(hist, degp.at[cid, sid, pl.ds(r * PAD_N, PAD_N)])


def _sc_aggregate(xm0, xm1, xm2, ei0, ei1, ei2):
    mesh = plsc.VectorSubcoreMesh(core_axis_name="c", subcore_axis_name="s")
    kern = pl.kernel(
        _sc_body,
        out_type=(
            jax.ShapeDtypeStruct((NC, R, PAD_N, H), jnp.float32),
            jax.ShapeDtypeStruct((NC, NS, R * PAD_N), jnp.float32),
        ),
        mesh=mesh,
        compiler_params=pltpu.CompilerParams(needs_layout_passes=False),
        scratch_types=[
            pltpu.VMEM((BPB, CHUNK), jnp.int32),
            pltpu.VMEM((BPB, CHUNK), jnp.int32),
            pltpu.VMEM((BPB, CHUNK), jnp.int32),
            pltpu.VMEM((BPB, CHUNK), jnp.int32),
            pltpu.VMEM((CHUNK, H), jnp.float32),
            pltpu.VMEM((CHUNK, H), jnp.float32),
            pltpu.VMEM((CHUNK, H), jnp.float32),
            pltpu.VMEM((PAD_N,), jnp.float32),
            pltpu.SemaphoreType.DMA,
            pltpu.SemaphoreType.DMA,
            pltpu.SemaphoreType.DMA,
            pltpu.SemaphoreType.DMA,
            pltpu.SemaphoreType.DMA,
            pltpu.SemaphoreType.DMA,
            pltpu.SemaphoreType.DMA,
            pltpu.SemaphoreType.DMA,
            pltpu.SemaphoreType.DMA,
            pltpu.SemaphoreType.DMA,
            pltpu.SemaphoreType.DMA,
            pltpu.VMEM_SHARED((PAD_N, H), jnp.float32),
        ],
    )
    return kern(xm0, xm1, xm2, ei0, ei1, ei2)


def _head_body(aggp_ref, degp_ref, bg_ref, w1_ref, b1_ref, wo_ref, bo_ref,
               out_ref):
    deg = jnp.sum(degp_ref[...], axis=(0, 1))
    h1 = jnp.zeros((HB, H), jnp.float32)
    for r in range(R):
        a = aggp_ref[0, r] + aggp_ref[1, r]
        w = 1.0 / jnp.maximum(deg[r].reshape(HB), 1.0)
        h1 = h1 + a * w[:, None] + bg_ref[r][None, :]
    h2 = jnp.maximum(h1, 0.0)
    z = jnp.dot(h2, w1_ref[...], preferred_element_type=jnp.float32)
    z = z + b1_ref[0][None, :]
    h3 = 1.0 / (1.0 + jnp.exp(-z))
    o = jnp.dot(h3, wo_ref[...], preferred_element_type=jnp.float32)
    out_ref[...] = o + bo_ref[0][None, :]


def _tc_head(aggp, degp5, b_gcn, W_nn1, b_nn1, W_out_p, b_out_p):
    return pl.pallas_call(
        _head_body,
        grid=(PAD_N // HB,),
        in_specs=[
            pl.BlockSpec((NC, R, HB, H), lambda i: (0, 0, i, 0)),
            pl.BlockSpec((NC, NS, R, HB // 128, 128), lambda i: (0, 0, 0, i, 0)),
            pl.BlockSpec((R, H), lambda i: (0, 0)),
            pl.BlockSpec((H, H), lambda i: (0, 0)),
            pl.BlockSpec((1, H), lambda i: (0, 0)),
            pl.BlockSpec((H, 128), lambda i: (0, 0)),
            pl.BlockSpec((1, 128), lambda i: (0, 0)),
        ],
        out_specs=pl.BlockSpec((HB, 128), lambda i: (i, 0)),
        out_shape=jax.ShapeDtypeStruct((PAD_N, 128), jnp.float32),
    )(aggp, degp5, b_gcn, W_nn1, b_nn1, W_out_p, b_out_p)


def kernel(features, edge_index_r0, edge_index_r1, edge_index_r2,
           W_gcn, b_gcn, W_nn1, b_nn1, W_out, b_out):
    xpad = jnp.pad(features, ((0, PAD_N - N), (0, 0)))
    xm0, xm1, xm2 = _tc_matmul(xpad, W_gcn)

    ei0 = edge_index_r0.reshape(2, NC * NS, NBLK, BPB, CHUNK)
    ei1 = edge_index_r1.reshape(2, NC * NS, NBLK, BPB, CHUNK)
    ei2 = edge_index_r2.reshape(2, NC * NS, NBLK, BPB, CHUNK)
    aggp, degp = _sc_aggregate(xm0, xm1, xm2, ei0, ei1, ei2)

    degp5 = degp.reshape(NC, NS, R, NB, 128)
    W_out_p = jnp.zeros((H, 128), jnp.float32).at[:, :2].set(W_out)
    b_out_p = jnp.zeros((1, 128), jnp.float32).at[0, :2].set(b_out)
    out = _tc_head(aggp, degp5, b_gcn, W_nn1, b_nn1.reshape(1, H),
                   W_out_p, b_out_p)
    return out[:N, :2]
